# baseline (speedup 1.0000x reference)
import jax
import jax.numpy as jnp
from jax import lax
from jax.experimental import pallas as pl
from jax.experimental.pallas import tpu as pltpu
from jax.experimental.pallas import tpu_sc as plsc

_SCALE = 64.0
_MARGIN = 0.35
_B = 1024
_V = 100000
_BR = 32

_NC = 2
_NS = 16
_NW = _NC * _NS
_BPW = _B // _NW


def _sc_gather_body(flat_hbm, label_hbm, t_hbm, lab_v, idx_v, t_v, sem):
    wid = lax.axis_index("s") * _NC + lax.axis_index("c")
    base = wid * _BPW
    pltpu.sync_copy(label_hbm.at[pl.ds(base, _BPW)], lab_v)
    row0 = base * _V
    for j in range(_BPW // 16):
        lab = lab_v[pl.ds(j * 16, 16)]
        idx = lab + (lax.iota(jnp.int32, 16) * _V + (row0 + j * 16 * _V))
        idx_v[pl.ds(j * 16, 16)] = idx
    pltpu.async_copy(flat_hbm.at[idx_v], t_v, sem).wait()
    pltpu.sync_copy(t_v, t_hbm.at[pl.ds(base, _BPW)])


def _sc_gather(cosine, label):
    mesh = plsc.VectorSubcoreMesh(core_axis_name="c", subcore_axis_name="s")
    return pl.kernel(
        _sc_gather_body,
        out_type=jax.ShapeDtypeStruct((_B,), jnp.float32),
        mesh=mesh,
        scratch_types=[
            pltpu.VMEM((_BPW,), jnp.int32),
            pltpu.VMEM((_BPW,), jnp.int32),
            pltpu.VMEM((_BPW,), jnp.float32),
            pltpu.SemaphoreType.DMA,
        ],
    )(cosine.reshape(_B * _V), label.astype(jnp.int32))


_CB = 8
_NBUF = 8
_NCH = _B // _CB
_LOG2E = 1.4426950408889634
_C2 = _SCALE * _LOG2E


def _tc_stream_body(t_ref, cos_hbm, out_ref, *scratch):
    bufs = scratch[:_NBUF]
    sems = scratch[_NBUF:]

    def issue(chunk, b):
        r0 = pl.multiple_of(chunk * _CB, _CB)
        pltpu.make_async_copy(cos_hbm.at[pl.ds(r0, _CB)], bufs[b],
                              sems[b]).start()

    def chunk_nll_sum(chunk, b):
        pltpu.make_async_copy(cos_hbm.at[pl.ds(0, _CB)], bufs[b],
                              sems[b]).wait()
        x = bufs[b][...]
        s = jnp.sum(jnp.exp2(x * _C2), axis=1, keepdims=True)
        r0 = pl.multiple_of(chunk * _CB, _CB)
        t64 = t_ref[pl.ds(r0, _CB), :] * _SCALE
        delta = _SCALE * _MARGIN
        sp = s - jnp.exp(t64) * (1.0 - jnp.exp(jnp.float32(-delta)))
        nll = jnp.log(sp) - t64 + delta
        return jnp.sum(nll, keepdims=True)

    for b in range(_NBUF):
        issue(b, b)

    def outer(i, acc):
        for b in range(_NBUF):
            chunk = i * _NBUF + b
            acc = acc + chunk_nll_sum(chunk, b)
            nxt = chunk + _NBUF

            @pl.when(nxt < _NCH)
            def _():
                issue(nxt, b)
        return acc

    acc = lax.fori_loop(0, _NCH // _NBUF, outer,
                        jnp.zeros((1, 1), jnp.float32))
    out_ref[...] = acc * (1.0 / _B)


def _tc_loss(t, cosine):
    out = pl.pallas_call(
        _tc_stream_body,
        in_specs=[
            pl.BlockSpec((_B, 1), lambda: (0, 0)),
            pl.BlockSpec(memory_space=pl.ANY),
        ],
        out_specs=pl.BlockSpec((1, 1), lambda: (0, 0)),
        out_shape=jax.ShapeDtypeStruct((1, 1), jnp.float32),
        scratch_shapes=(
            [pltpu.VMEM((_CB, _V), jnp.float32) for _ in range(_NBUF)]
            + [pltpu.SemaphoreType.DMA for _ in range(_NBUF)]
        ),
    )(t.reshape(_B, 1), cosine)
    return out[0, 0]


def kernel(cosine, label):
    t = _sc_gather(cosine, label)
    return _tc_loss(t, cosine)


# speedup vs baseline: 1.0083x; 1.0083x over previous
import jax
import jax.numpy as jnp
from jax import lax
from jax.experimental import pallas as pl
from jax.experimental.pallas import tpu as pltpu
from jax.experimental.pallas import tpu_sc as plsc

_SCALE = 64.0
_MARGIN = 0.35
_B = 1024
_V = 100000

_NC = 2
_NS = 16
_NW = _NC * _NS
_RPW = _B // _NW

_CHUNK = 10000
_CPR = _V // _CHUNK
_NCHW = _RPW * _CPR
_NBUF = 4
_UNR = 5
_ILOOP = _CHUNK // (16 * _UNR)


def _sc_main_body(flat_hbm, label_hbm, s_hbm, t_hbm,
                  lab_v, idx_v, t_v, s_v, *bufsems):
    bufs = bufsems[:_NBUF]
    sems = bufsems[_NBUF:2 * _NBUF]
    gsem = bufsems[2 * _NBUF]
    wid = lax.axis_index("s") * _NC + lax.axis_index("c")
    base = wid * _RPW
    flat0 = base * _V

    pltpu.sync_copy(label_hbm.at[pl.ds(base, _RPW)], lab_v)
    for j in range(_RPW // 16):
        lab = lab_v[pl.ds(j * 16, 16)]
        idx = lab + (lax.iota(jnp.int32, 16) * _V + (flat0 + j * 16 * _V))
        idx_v[pl.ds(j * 16, 16)] = idx
    gather = pltpu.async_copy(flat_hbm.at[idx_v], t_v, gsem)

    for j in range(_RPW):
        s_v[pl.ds(j * 16, 16)] = jnp.zeros((16,), jnp.float32)

    def issue(g, b):
        off = flat0 + g * _CHUNK
        pltpu.async_copy(flat_hbm.at[pl.ds(off, _CHUNK)], bufs[b], sems[b])

    def chunk_sum(b):
        def vbody(k, carry):
            a0, a1 = carry
            off = k * (16 * _UNR)
            for u in range(_UNR):
                v = bufs[b][pl.ds(off + u * 16, 16)]
                e = jnp.exp(v * _SCALE)
                if u % 2 == 0:
                    a0 = a0 + e
                else:
                    a1 = a1 + e
            return (a0, a1)

        z = jnp.zeros((16,), jnp.float32)
        a0, a1 = lax.fori_loop(0, _ILOOP, vbody, (z, z))
        return a0 + a1

    for b in range(_NBUF):
        issue(b, b)

    def outer(i, carry):
        for b in range(_NBUF):
            g = i * _NBUF + b
            pltpu.make_async_copy(flat_hbm.at[pl.ds(0, _CHUNK)], bufs[b],
                                  sems[b]).wait()
            part = chunk_sum(b)
            sl = pl.ds((g // _CPR) * 16, 16)
            s_v[sl] = s_v[sl] + part
            nxt = g + _NBUF

            @pl.when(nxt < _NCHW)
            def _():
                issue(nxt, b)
        return carry

    lax.fori_loop(0, _NCHW // _NBUF, outer, jnp.int32(0))

    gather.wait()
    pltpu.sync_copy(s_v, s_hbm.at[pl.ds(base * 16, _RPW * 16)])
    pltpu.sync_copy(t_v, t_hbm.at[pl.ds(base, _RPW)])


def _sc_main(cosine, label):
    mesh = plsc.VectorSubcoreMesh(core_axis_name="c", subcore_axis_name="s")
    return pl.kernel(
        _sc_main_body,
        out_type=[jax.ShapeDtypeStruct((_B * 16,), jnp.float32),
                  jax.ShapeDtypeStruct((_B,), jnp.float32)],
        mesh=mesh,
        scratch_types=(
            [pltpu.VMEM((_RPW,), jnp.int32),
             pltpu.VMEM((_RPW,), jnp.int32),
             pltpu.VMEM((_RPW,), jnp.float32),
             pltpu.VMEM((_RPW * 16,), jnp.float32)]
            + [pltpu.VMEM((_CHUNK,), jnp.float32) for _ in range(_NBUF)]
            + [pltpu.SemaphoreType.DMA for _ in range(_NBUF + 1)]
        ),
    )(cosine.reshape(_B * _V), label.astype(jnp.int32))


def _tc_combine_body(s_ref, t_ref, out_ref):
    s = jnp.sum(s_ref[...], axis=1, keepdims=True)
    t64 = t_ref[...] * _SCALE
    delta = _SCALE * _MARGIN
    sp = s - jnp.exp(t64) * (1.0 - jnp.exp(jnp.float32(-delta)))
    nll = jnp.log(sp) - t64 + delta
    out_ref[...] = jnp.sum(nll, keepdims=True) * (1.0 / _B)


def _tc_combine(s, t):
    out = pl.pallas_call(
        _tc_combine_body,
        in_specs=[
            pl.BlockSpec((_B, 16), lambda: (0, 0)),
            pl.BlockSpec((_B, 1), lambda: (0, 0)),
        ],
        out_specs=pl.BlockSpec((1, 1), lambda: (0, 0)),
        out_shape=jax.ShapeDtypeStruct((1, 1), jnp.float32),
    )(s.reshape(_B, 16), t.reshape(_B, 1))
    return out[0, 0]


def kernel(cosine, label):
    s, t = _sc_main(cosine, label)
    return _tc_combine(s, t)


# speedup vs baseline: 1.5853x; 1.5722x over previous
import jax
import jax.numpy as jnp
from jax import lax
from jax.experimental import pallas as pl
from jax.experimental.pallas import tpu as pltpu
from jax.experimental.pallas import tpu_sc as plsc

_SCALE = 64.0
_MARGIN = 0.35
_B = 1024
_V = 100000
_VMAIN = 99968
_LOG2E = 1.4426950408889634
_C2 = _SCALE * _LOG2E

_NC = 2
_NS = 16
_NW = _NC * _NS
_RPW = _B // _NW
_TRW = _RPW // 8

_CW = 1408
_CPT = _VMAIN // _CW
_NCHW = _TRW * _CPT
_NBUF = 4
_UNR = 4
_ILOOP = _CW // (16 * _UNR)


def _sc_main_body(cos_hbm, label_hbm, s_hbm, t_hbm,
                  lab_v, t_v, s_v, *bufsems):
    bufs = bufsems[:_TRW]
    sems = bufsems[_TRW:2 * _TRW]
    wid = lax.axis_index("s") * _NC + lax.axis_index("c")
    base = wid * _RPW
    iota16 = lax.iota(jnp.int32, 16)
    zeros16 = jnp.zeros((16,), jnp.float32)

    pltpu.sync_copy(label_hbm.at[pl.ds(base, _RPW)], lab_v)
    for j in range(_RPW):
        s_v[pl.ds(j * 16, 16)] = zeros16
    for j in range(_RPW // 16):
        t_v[pl.ds(j * 16, 16)] = zeros16

    lab_cc, lab_go, lab_lane = [], [], []
    for rl in range(_RPW):
        grp = lab_v[pl.ds((rl // 16) * 16, 16)]
        lab_s = grp[rl % 16]
        o_lab = lab_s % _CW
        go = (o_lab // 16) * 16
        lab_cc.append(lab_s // _CW)
        lab_go.append(go)
        lab_lane.append(o_lab - go)

    def issue(cc, tr):
        pltpu.async_copy(
            cos_hbm.at[pl.ds(base + tr * 8, 8), pl.ds(cc * _CW, _CW)],
            bufs[tr], sems[tr])

    for tr in range(_TRW):
        issue(0, tr)

    def outer(cc, carry):
        for tr in range(_TRW):
            pltpu.make_async_copy(
                cos_hbm.at[pl.ds(0, 8), pl.ds(0, _CW)], bufs[tr],
                sems[tr]).wait()
            for r in range(8):
                def vbody(k, ab, tr=tr, r=r):
                    a0, a1 = ab
                    off = k * (16 * _UNR)
                    for u in range(_UNR):
                        v = bufs[tr][r, pl.ds(off + u * 16, 16)]
                        e = jnp.exp(v * _SCALE)
                        if u % 2 == 0:
                            a0 = a0 + e
                        else:
                            a1 = a1 + e
                    return (a0, a1)

                a0, a1 = lax.fori_loop(0, _ILOOP, vbody, (zeros16, zeros16))
                rl = tr * 8 + r
                sl = pl.ds(rl * 16, 16)
                s_v[sl] = s_v[sl] + (a0 + a1)

                @pl.when(cc == lab_cc[rl])
                def _(tr=tr, r=r, rl=rl):
                    vec = bufs[tr][r, pl.ds(lab_go[rl], 16)]
                    idxv = jnp.zeros((16,), jnp.int32) + lab_lane[rl]
                    tv = lax.gather(
                        vec, idxv[:, None],
                        lax.GatherDimensionNumbers(
                            offset_dims=(), collapsed_slice_dims=(0,),
                            start_index_map=(0,)),
                        slice_sizes=(1,),
                        mode=lax.GatherScatterMode.PROMISE_IN_BOUNDS)
                    tsl = pl.ds((rl // 16) * 16, 16)
                    t_v[tsl] = t_v[tsl] + jnp.where(iota16 == (rl % 16),
                                                    tv, 0.0)

            @pl.when(cc + 1 < _CPT)
            def _(tr=tr):
                issue(cc + 1, tr)
        return carry

    lax.fori_loop(0, _CPT, outer, jnp.int32(0))

    pltpu.sync_copy(s_v, s_hbm.at[pl.ds(base * 16, _RPW * 16)])
    pltpu.sync_copy(t_v, t_hbm.at[pl.ds(base, _RPW)])


def _sc_main(cosine, label):
    mesh = plsc.VectorSubcoreMesh(core_axis_name="c", subcore_axis_name="s")
    return pl.kernel(
        _sc_main_body,
        out_type=[jax.ShapeDtypeStruct((_B * 16,), jnp.float32),
                  jax.ShapeDtypeStruct((_B,), jnp.float32)],
        mesh=mesh,
        scratch_types=(
            [pltpu.VMEM((_RPW,), jnp.int32),
             pltpu.VMEM((_RPW,), jnp.float32),
             pltpu.VMEM((_RPW * 16,), jnp.float32)]
            + [pltpu.VMEM((8, _CW), jnp.float32) for _ in range(_TRW)]
            + [pltpu.SemaphoreType.DMA for _ in range(_TRW)]
        ),
    )(cosine, label.astype(jnp.int32))


def _tc_combine_body(s_ref, t_ref, lab_ref, tail_ref, out_ref):
    tail = tail_ref[...]
    col = lax.broadcasted_iota(jnp.int32, tail.shape, 1) + _VMAIN
    valid = col < _V
    tail = jnp.where(valid, tail, -1.0)
    s_tail = jnp.sum(jnp.where(valid, jnp.exp2(tail * _C2), 0.0),
                     axis=1, keepdims=True)
    lab = lab_ref[...]
    t_tail = jnp.sum(jnp.where(col == lab, tail, 0.0), axis=1,
                     keepdims=True)

    s = jnp.sum(s_ref[...], axis=1, keepdims=True) + s_tail
    t64 = (t_ref[...] + t_tail) * _SCALE
    delta = _SCALE * _MARGIN
    sp = s - jnp.exp(t64) * (1.0 - jnp.exp(jnp.float32(-delta)))
    nll = jnp.log(sp) - t64 + delta
    out_ref[...] = jnp.sum(nll, keepdims=True) * (1.0 / _B)


def _tc_combine(s, t, label, cosine):
    out = pl.pallas_call(
        _tc_combine_body,
        grid=(1,),
        in_specs=[
            pl.BlockSpec((_B, 16), lambda i: (0, 0)),
            pl.BlockSpec((_B, 1), lambda i: (0, 0)),
            pl.BlockSpec((_B, 1), lambda i: (0, 0)),
            pl.BlockSpec((_B, 128), lambda i: (0, _VMAIN // 128)),
        ],
        out_specs=pl.BlockSpec((1, 1), lambda i: (0, 0)),
        out_shape=jax.ShapeDtypeStruct((1, 1), jnp.float32),
    )(s.reshape(_B, 16), t.reshape(_B, 1),
      label.astype(jnp.int32).reshape(_B, 1), cosine)
    return out[0, 0]


def kernel(cosine, label):
    s, t = _sc_main(cosine, label)
    return _tc_combine(s, t, label, cosine)
